# baseline (device time: 45580 ns/iter reference)
import jax
import jax.numpy as jnp
from jax import lax
from jax.experimental import pallas as pl
from jax.experimental.pallas import tpu as pltpu

N_DEV = 4
B, SQ, SKV, DH = 2, 512, 512, 64
H_LOC = 8
D_LOC = H_LOC * DH
HALF = D_LOC // 2
D_MODEL = 768
BLK = 64
N_HOP = N_DEV - 1

RING_B = (0, 0, 1, 1)
RING_HH = (0, 1, 0, 1)
RING_DIR = (+1, -1, +1, -1)


def kernel(x, Wq, K_ext, V_ext, Wo):
    def body(x_ref, wq_ref, k_ref, v_ref, wo_ref, out_ref, comm, ssem, rsem):
        my = lax.axis_index("i")
        left = lax.rem(my + N_DEV - 1, N_DEV)
        right = lax.rem(my + 1, N_DEV)

        barrier = pltpu.get_barrier_semaphore()
        for nbr in (left, right):
            pl.semaphore_signal(barrier, inc=1, device_id=(nbr,),
                                device_id_type=pl.DeviceIdType.MESH)
        pl.semaphore_wait(barrier, 2)

        xf = x_ref[...].reshape(B * SQ, D_MODEL).astype(jnp.bfloat16)
        wq = wq_ref[:, pl.ds(my * D_LOC, D_LOC)].astype(jnp.bfloat16)
        q = lax.dot_general(xf, wq, (((1,), (0,)), ((), ())),
                            preferred_element_type=jnp.float32)
        q = (q * 0.125).astype(jnp.bfloat16)

        qb = lax.broadcasted_iota(jnp.int32, (SQ, SKV), 0) // BLK
        kb = lax.broadcasted_iota(jnp.int32, (SQ, SKV), 1) // BLK
        mask = (qb == kb) | (kb == 0) | (lax.rem(qb + kb, 3) == 0)
        bias = jnp.where(mask, 0.0, -1e9)

        def attn_group(ring):
            b, hh = RING_B[ring], RING_HH[ring]
            for hi in range(4):
                h = hh * 4 + hi
                q_bh = q[b * SQ:(b + 1) * SQ, h * DH:(h + 1) * DH]
                k_bh = k_ref[b, :, h, :].astype(jnp.bfloat16)
                v_bh = v_ref[b, :, h, :].astype(jnp.bfloat16)
                s = lax.dot_general(q_bh, k_bh, (((1,), (1,)), ((), ())),
                                    preferred_element_type=jnp.float32)
                w = jnp.exp(s + bias)
                denom = jnp.sum(w, axis=1, keepdims=True)
                ctx = lax.dot_general(w.astype(jnp.bfloat16), v_bh,
                                      (((1,), (0,)), ((), ())),
                                      preferred_element_type=jnp.float32)
                ctx = (ctx * (1.0 / denom)).astype(jnp.bfloat16)
                comm[ring, 0, :, hi * DH:(hi + 1) * DH] = ctx

        rdmas = {}

        def start_hop(ring, hop):
            dev = right if RING_DIR[ring] > 0 else left
            rd = pltpu.make_async_remote_copy(
                src_ref=comm.at[ring, hop], dst_ref=comm.at[ring, hop + 1],
                send_sem=ssem.at[ring, hop], recv_sem=rsem.at[ring, hop],
                device_id=(dev,), device_id_type=pl.DeviceIdType.MESH)
            rd.start()
            rdmas[(ring, hop)] = rd

        def origin(ring, hop):
            if RING_DIR[ring] > 0:
                return lax.rem(my + N_DEV - 1 - hop, N_DEV)
            return lax.rem(my + 1 + hop, N_DEV)

        def wo_q(ring, dev_idx):
            off = RING_HH[ring] * HALF
            return wo_ref[pl.ds(dev_idx * D_LOC + off, HALF), :].astype(
                jnp.bfloat16)

        def qdot(ring, slot, dev_idx):
            return lax.dot_general(comm[ring, slot], wo_q(ring, dev_idx),
                                   (((1,), (0,)), ((), ())),
                                   preferred_element_type=jnp.float32)

        attn_group(0)
        start_hop(0, 0)
        attn_group(1)
        start_hop(1, 0)
        attn_group(2)
        start_hop(2, 0)
        rdmas[(0, 0)].wait_recv()
        start_hop(0, 1)
        attn_group(3)
        start_hop(3, 0)
        rdmas[(1, 0)].wait_recv()
        start_hop(1, 1)

        acc0 = qdot(0, 0, my) + qdot(1, 0, my)
        acc1 = qdot(2, 0, my) + qdot(3, 0, my)

        rdmas[(2, 0)].wait_recv()
        start_hop(2, 1)
        acc0 = acc0 + qdot(0, 1, origin(0, 0))
        rdmas[(0, 1)].wait_recv()
        start_hop(0, 2)
        acc0 = acc0 + qdot(1, 1, origin(1, 0))
        rdmas[(3, 0)].wait_recv()
        start_hop(3, 1)
        acc1 = acc1 + qdot(2, 1, origin(2, 0))
        rdmas[(1, 1)].wait_recv()
        start_hop(1, 2)
        acc1 = acc1 + qdot(3, 1, origin(3, 0))
        rdmas[(2, 1)].wait_recv()
        start_hop(2, 2)
        acc0 = acc0 + qdot(0, 2, origin(0, 1))
        rdmas[(0, 2)].wait_recv()
        acc0 = acc0 + qdot(1, 2, origin(1, 1))
        rdmas[(3, 1)].wait_recv()
        start_hop(3, 2)
        acc1 = acc1 + qdot(2, 2, origin(2, 1))
        rdmas[(1, 2)].wait_recv()
        acc1 = acc1 + qdot(3, 2, origin(3, 1))
        acc0 = acc0 + qdot(0, 3, origin(0, 2))
        rdmas[(2, 2)].wait_recv()
        acc0 = acc0 + qdot(1, 3, origin(1, 2))
        acc1 = acc1 + qdot(2, 3, origin(2, 2))
        rdmas[(3, 2)].wait_recv()
        acc1 = acc1 + qdot(3, 3, origin(3, 2))

        for key in rdmas:
            rdmas[key].wait_send()

        out_ref[0] = acc0
        out_ref[1] = acc1

    return pl.pallas_call(
        body,
        out_shape=jax.ShapeDtypeStruct((B, SQ, D_MODEL), jnp.float32),
        in_specs=[pl.BlockSpec(memory_space=pltpu.VMEM)] * 5,
        out_specs=pl.BlockSpec(memory_space=pltpu.VMEM),
        scratch_shapes=[
            pltpu.VMEM((4, N_DEV, SQ, HALF), jnp.bfloat16),
            pltpu.SemaphoreType.DMA((4, N_HOP)),
            pltpu.SemaphoreType.DMA((4, N_HOP)),
        ],
        compiler_params=pltpu.CompilerParams(collective_id=0),
    )(x, Wq, K_ext, V_ext, Wo)


# device time: 45106 ns/iter; 1.0105x vs baseline; 1.0105x over previous
import jax
import jax.numpy as jnp
from jax import lax
from jax.experimental import pallas as pl
from jax.experimental.pallas import tpu as pltpu

N_DEV = 4
B, SQ, SKV, DH = 2, 512, 512, 64
H_LOC = 8
D_LOC = H_LOC * DH
HALF = D_LOC // 2
D_MODEL = 768
BLK = 64
N_HOP = N_DEV - 1

RING_B = (0, 0, 1, 1)
RING_HH = (0, 1, 0, 1)
RING_DIR = (+1, -1, +1, -1)


def kernel(x, Wq, K_ext, V_ext, Wo):
    def body(x_ref, wq_ref, k_ref, v_ref, wo_ref, out_ref, comm, ssem, rsem):
        my = lax.axis_index("i")
        left = lax.rem(my + N_DEV - 1, N_DEV)
        right = lax.rem(my + 1, N_DEV)

        barrier = pltpu.get_barrier_semaphore()
        for nbr in (left, right):
            pl.semaphore_signal(barrier, inc=1, device_id=(nbr,),
                                device_id_type=pl.DeviceIdType.MESH)
        pl.semaphore_wait(barrier, 2)

        xf = x_ref[...].reshape(B * SQ, D_MODEL).astype(jnp.bfloat16)
        wq = wq_ref[:, pl.ds(my * D_LOC, D_LOC)].astype(jnp.bfloat16)
        q = lax.dot_general(xf, wq, (((1,), (0,)), ((), ())),
                            preferred_element_type=jnp.float32)
        q = (q * 0.125).astype(jnp.bfloat16)

        qb = lax.broadcasted_iota(jnp.int32, (SQ, SKV), 0) // BLK
        kb = lax.broadcasted_iota(jnp.int32, (SQ, SKV), 1) // BLK
        mask = (qb == kb) | (kb == 0) | (lax.rem(qb + kb, 3) == 0)
        bias = jnp.where(mask, 0.0, -1e9).astype(jnp.bfloat16)

        def attn_group(ring):
            b, hh = RING_B[ring], RING_HH[ring]
            for hi in range(4):
                h = hh * 4 + hi
                q_bh = q[b * SQ:(b + 1) * SQ, h * DH:(h + 1) * DH]
                k_bh = k_ref[b, :, h, :].astype(jnp.bfloat16)
                v_bh = v_ref[b, :, h, :].astype(jnp.bfloat16)
                s = lax.dot_general(q_bh, k_bh, (((1,), (1,)), ((), ())),
                                    preferred_element_type=jnp.float32)
                w = jnp.exp(s.astype(jnp.bfloat16) + bias)
                denom = jnp.sum(w, axis=1, keepdims=True,
                                dtype=jnp.float32)
                ctx = lax.dot_general(w, v_bh,
                                      (((1,), (0,)), ((), ())),
                                      preferred_element_type=jnp.float32)
                ctx = (ctx * (1.0 / denom)).astype(jnp.bfloat16)
                comm[ring, 0, :, hi * DH:(hi + 1) * DH] = ctx

        rdmas = {}

        def start_hop(ring, hop):
            dev = right if RING_DIR[ring] > 0 else left
            rd = pltpu.make_async_remote_copy(
                src_ref=comm.at[ring, hop], dst_ref=comm.at[ring, hop + 1],
                send_sem=ssem.at[ring, hop], recv_sem=rsem.at[ring, hop],
                device_id=(dev,), device_id_type=pl.DeviceIdType.MESH)
            rd.start()
            rdmas[(ring, hop)] = rd

        def origin(ring, hop):
            if RING_DIR[ring] > 0:
                return lax.rem(my + N_DEV - 1 - hop, N_DEV)
            return lax.rem(my + 1 + hop, N_DEV)

        def wo_q(ring, dev_idx):
            off = RING_HH[ring] * HALF
            return wo_ref[pl.ds(dev_idx * D_LOC + off, HALF), :].astype(
                jnp.bfloat16)

        def qdot(ring, slot, dev_idx):
            return lax.dot_general(comm[ring, slot], wo_q(ring, dev_idx),
                                   (((1,), (0,)), ((), ())),
                                   preferred_element_type=jnp.float32)

        attn_group(0)
        start_hop(0, 0)
        attn_group(1)
        start_hop(1, 0)
        attn_group(2)
        start_hop(2, 0)
        rdmas[(0, 0)].wait_recv()
        start_hop(0, 1)
        attn_group(3)
        start_hop(3, 0)
        rdmas[(1, 0)].wait_recv()
        start_hop(1, 1)

        acc0 = qdot(0, 0, my) + qdot(1, 0, my)
        acc1 = qdot(2, 0, my) + qdot(3, 0, my)

        rdmas[(2, 0)].wait_recv()
        start_hop(2, 1)
        acc0 = acc0 + qdot(0, 1, origin(0, 0))
        rdmas[(0, 1)].wait_recv()
        start_hop(0, 2)
        acc0 = acc0 + qdot(1, 1, origin(1, 0))
        rdmas[(3, 0)].wait_recv()
        start_hop(3, 1)
        acc1 = acc1 + qdot(2, 1, origin(2, 0))
        rdmas[(1, 1)].wait_recv()
        start_hop(1, 2)
        acc1 = acc1 + qdot(3, 1, origin(3, 0))
        rdmas[(2, 1)].wait_recv()
        start_hop(2, 2)
        acc0 = acc0 + qdot(0, 2, origin(0, 1))
        rdmas[(0, 2)].wait_recv()
        acc0 = acc0 + qdot(1, 2, origin(1, 1))
        rdmas[(3, 1)].wait_recv()
        start_hop(3, 2)
        acc1 = acc1 + qdot(2, 2, origin(2, 1))
        rdmas[(1, 2)].wait_recv()
        acc1 = acc1 + qdot(3, 2, origin(3, 1))
        acc0 = acc0 + qdot(0, 3, origin(0, 2))
        rdmas[(2, 2)].wait_recv()
        acc0 = acc0 + qdot(1, 3, origin(1, 2))
        acc1 = acc1 + qdot(2, 3, origin(2, 2))
        rdmas[(3, 2)].wait_recv()
        acc1 = acc1 + qdot(3, 3, origin(3, 2))

        for key in rdmas:
            rdmas[key].wait_send()

        out_ref[0] = acc0
        out_ref[1] = acc1

    return pl.pallas_call(
        body,
        out_shape=jax.ShapeDtypeStruct((B, SQ, D_MODEL), jnp.float32),
        in_specs=[pl.BlockSpec(memory_space=pltpu.VMEM)] * 5,
        out_specs=pl.BlockSpec(memory_space=pltpu.VMEM),
        scratch_shapes=[
            pltpu.VMEM((4, N_DEV, SQ, HALF), jnp.bfloat16),
            pltpu.SemaphoreType.DMA((4, N_HOP)),
            pltpu.SemaphoreType.DMA((4, N_HOP)),
        ],
        compiler_params=pltpu.CompilerParams(collective_id=0),
    )(x, Wq, K_ext, V_ext, Wo)
